# baseline (speedup 1.0000x reference)
import functools

import jax
import jax.numpy as jnp
from jax import lax
from jax.experimental import pallas as pl
from jax.experimental.pallas import tpu as pltpu
from jax.experimental.pallas import tpu_sc as plsc

_NC = 2
_NS = 16
_NW = _NC * _NS


@functools.partial(jax.jit, static_argnums=(2,))
def _gather(X, Wp, D):
    S, H = X.shape
    s_per_w = S // _NW
    mesh = plsc.VectorSubcoreMesh(core_axis_name="c", subcore_axis_name="s")

    @functools.partial(
        pl.kernel,
        mesh=mesh,
        out_type=jax.ShapeDtypeStruct((S, H, 2 * D), jnp.float32),
        scratch_types=[
            pltpu.VMEM((s_per_w, H), jnp.int32),
            pltpu.VMEM((H, 2 * D), jnp.float32),
            pltpu.VMEM((H, 2 * D), jnp.float32),
            pltpu.SemaphoreType.DMA,
            pltpu.SemaphoreType.DMA,
        ],
    )
    def body(idx_hbm, table_hbm, out_hbm, idx_v, buf0, buf1, sem0, sem1):
        wid = lax.axis_index("s") * _NC + lax.axis_index("c")
        base = wid * s_per_w
        pltpu.sync_copy(idx_hbm.at[pl.ds(base, s_per_w)], idx_v)

        pltpu.async_copy(table_hbm.at[idx_v.at[0]], buf0, sem0)

        def pair(g, carry):
            c0 = 2 * g
            pltpu.async_copy(table_hbm.at[idx_v.at[c0 + 1]], buf1, sem1)
            pltpu.make_async_copy(table_hbm.at[idx_v.at[c0]], buf0, sem0).wait()
            pltpu.sync_copy(buf0, out_hbm.at[base + c0])

            @pl.when(g + 1 < s_per_w // 2)
            def _():
                pltpu.async_copy(table_hbm.at[idx_v.at[c0 + 2]], buf0, sem0)

            pltpu.make_async_copy(
                table_hbm.at[idx_v.at[c0 + 1]], buf1, sem1).wait()
            pltpu.sync_copy(buf1, out_hbm.at[base + c0 + 1])
            return carry

        lax.fori_loop(0, s_per_w // 2, pair, 0)

    return body(X, Wp)


def kernel(X, W):
    D = W.shape[1]
    M = jnp.eye(D, 2 * D, dtype=jnp.float32)
    Wp = jnp.dot(W, M, precision=jax.lax.Precision.DEFAULT)
    out = _gather(X.astype(jnp.int32), Wp, D)
    return out[:, :, :D]
